# Initial kernel scaffold; baseline (speedup 1.0000x reference)
import jax
import jax.numpy as jnp
from jax.experimental import pallas as pl


def kernel(x):
    raise NotImplementedError("write your pallas kernel here")


# speedup vs baseline: 23.8591x; 23.8591x over previous
import functools
import jax
import jax.numpy as jnp
from jax.experimental import pallas as pl
from jax.experimental.pallas import tpu as pltpu

_FRAC = 0.36787944117144233


def _gate_body(k_const, x_ref, o_ref, bits_ref):
    bits = jax.lax.bitcast_convert_type(x_ref[...], jnp.int32) & jnp.int32(
        0x7FFFFFFF
    )
    bits_ref[...] = bits

    def step(i, p):
        cand = p | (jnp.int32(1) << (jnp.int32(30) - i))
        cnt = jnp.sum((bits_ref[...] >= cand).astype(jnp.int32))
        return jax.lax.select(cnt >= jnp.int32(k_const), cand, p)

    p = jax.lax.fori_loop(0, 31, step, jnp.int32(0))
    o_ref[...] = jnp.where(bits_ref[...] >= p, x_ref[...], jnp.float32(0.0))


def kernel(x):
    n = x.size
    k = max(1, int(n * _FRAC))
    if k >= n:
        return x
    return pl.pallas_call(
        functools.partial(_gate_body, k),
        out_shape=jax.ShapeDtypeStruct(x.shape, x.dtype),
        scratch_shapes=[pltpu.VMEM(x.shape, jnp.int32)],
    )(x)
